# Initial kernel scaffold; baseline (speedup 1.0000x reference)
import jax
import jax.numpy as jnp
from jax.experimental import pallas as pl


def kernel(prob, label):
    raise NotImplementedError("write your pallas kernel here")


# speedup vs baseline: 1.3265x; 1.3265x over previous
import functools
import jax
import jax.numpy as jnp
from jax import lax
from jax.experimental import pallas as pl
from jax.experimental.pallas import tpu as pltpu
from jax.experimental.pallas import tpu_sc as plsc

TOPK = 5
BATCH = 1024
NCLS = 100000
NWORK = 32
ROWS_W = BATCH // NWORK
NSLICE = NCLS // 16


def _reck_body(probf_hbm, label_hbm, out_hbm, row_v, lab_v, val_v, out_v, sem):
    cid = lax.axis_index("c")
    sid = lax.axis_index("s")
    wid = sid * 2 + cid
    row0 = wid * ROWS_W

    pltpu.sync_copy(label_hbm.at[pl.ds(row0, ROWS_W)], lab_v)
    iota = lax.iota(jnp.int32, 16)
    for h in range(2):
        lab16 = lab_v[pl.ds(h * 16, 16)]
        rows16 = (row0 + h * 16) + iota
        idx16 = rows16 * NCLS + lab16
        pltpu.async_copy(probf_hbm.at[idx16], val_v.at[pl.ds(h * 16, 16)], sem).wait()

    def ge_body(i, acc):
        x = row_v[pl.ds(i * 16, 16)]
        return acc[0] + jnp.where(x >= acc[1], 1.0, 0.0), acc[1]

    def gt_body(i, acc):
        x = row_v[pl.ds(i * 16, 16)]
        return acc[0] + jnp.where(x > acc[1], 1.0, 0.0), acc[1]

    def row_body(r, recall_vec):
        base = (row0 + r) * NCLS
        pltpu.sync_copy(probf_hbm.at[pl.ds(base, NCLS)], row_v)
        r16 = jnp.full((16,), r, jnp.int32)
        v_b = plsc.load_gather(val_v, [r16])
        lab_b = plsc.load_gather(lab_v, [r16])
        lab = jnp.max(lab_b)
        s_lab = lab // 16

        acc = jnp.zeros((16,), jnp.float32)
        acc, _ = lax.fori_loop(0, s_lab, ge_body, (acc, v_b))
        xm = row_v[pl.ds(s_lab * 16, 16)]
        pre = (s_lab * 16 + iota) < lab
        acc = acc + jnp.where(jnp.where(pre, xm >= v_b, xm > v_b), 1.0, 0.0)
        acc, _ = lax.fori_loop(s_lab + 1, NSLICE, gt_body, (acc, v_b))

        rank = jnp.sum(acc)
        hit = jnp.logical_and(rank < float(TOPK), lab != 0)
        return recall_vec + jnp.where(hit, 1.0, 0.0)

    recall_vec = lax.fori_loop(0, ROWS_W, row_body, jnp.zeros((16,), jnp.float32))
    out_v[...] = recall_vec
    pltpu.sync_copy(out_v, out_hbm.at[wid])


def kernel(prob, label):
    probf = prob.reshape(-1)
    mesh = plsc.VectorSubcoreMesh(core_axis_name="c", subcore_axis_name="s")
    run = functools.partial(
        pl.kernel,
        mesh=mesh,
        compiler_params=pltpu.CompilerParams(needs_layout_passes=False),
        out_type=jax.ShapeDtypeStruct((NWORK, 16), jnp.float32),
        scratch_types=[
            pltpu.VMEM((NCLS,), jnp.float32),
            pltpu.VMEM((ROWS_W,), jnp.int32),
            pltpu.VMEM((ROWS_W,), jnp.float32),
            pltpu.VMEM((16,), jnp.float32),
            pltpu.SemaphoreType.DMA,
        ],
    )(_reck_body)
    parts = run(probf, label)
    return parts[:, 0].sum() / jnp.float32(BATCH)


# speedup vs baseline: 2.4926x; 1.8791x over previous
import functools
import jax
import jax.numpy as jnp
from jax import lax
from jax.experimental import pallas as pl
from jax.experimental.pallas import tpu as pltpu
from jax.experimental.pallas import tpu_sc as plsc

TOPK = 5
BATCH = 1024
NCLS = 100000
NWORK = 32
ROWS_W = BATCH // NWORK
CH = 50000
SCH = CH // 16
UNROLL = 25
NACC = 5


def _count_chunk(buf, b0s, lab, s_lab, v_b, iota):
    p_end = jnp.clip(s_lab - b0s, 0, SCH)
    g_start = jnp.clip(s_lab + 1 - b0s, 0, SCH)

    def mk_block(ge):
        def blk(i, accs):
            base = i * UNROLL
            accs = list(accs)
            for k in range(UNROLL):
                x = buf[pl.ds((base + k) * 16, 16)]
                cond = (x >= v_b) if ge else (x > v_b)
                accs[k % NACC] = accs[k % NACC] + jnp.where(cond, 1.0, 0.0)
            return tuple(accs)
        return blk

    def mk_one(ge):
        def one(i, acc):
            x = buf[pl.ds(i * 16, 16)]
            cond = (x >= v_b) if ge else (x > v_b)
            return acc + jnp.where(cond, 1.0, 0.0)
        return one

    zeros = jnp.zeros((16,), jnp.float32)
    accs = (zeros,) * NACC
    nfull = p_end // UNROLL
    accs = lax.fori_loop(0, nfull, mk_block(True), accs)
    rem = lax.fori_loop(nfull * UNROLL, p_end, mk_one(True), zeros)
    m = s_lab - b0s
    mc = jnp.clip(m, 0, SCH - 1)
    xm = buf[pl.ds(mc * 16, 16)]
    gidx = (b0s + mc) * 16 + iota
    mix = jnp.where(jnp.where(gidx < lab, xm >= v_b, xm > v_b), 1.0, 0.0)
    inr = jnp.logical_and(m >= 0, m < SCH)
    rem = rem + jnp.where(inr, mix, 0.0)
    g_align = jnp.minimum((g_start + UNROLL - 1) // UNROLL * UNROLL, SCH)
    rem = lax.fori_loop(g_start, g_align, mk_one(False), rem)
    accs = lax.fori_loop(g_align // UNROLL, SCH // UNROLL, mk_block(False), accs)
    total = rem
    for a in accs:
        total = total + a
    return total


def _reck_body(probf_hbm, label_hbm, out_hbm,
               bufA, bufB, lab_v, val_v, out_v, semA, semB):
    cid = lax.axis_index("c")
    sid = lax.axis_index("s")
    wid = sid * 2 + cid
    row0 = wid * ROWS_W
    word0 = row0 * NCLS

    pltpu.sync_copy(label_hbm.at[pl.ds(row0, ROWS_W)], lab_v)
    iota = lax.iota(jnp.int32, 16)
    for h in range(2):
        lab16 = lab_v[pl.ds(h * 16, 16)]
        rows16 = (row0 + h * 16) + iota
        idx16 = rows16 * NCLS + lab16
        pltpu.async_copy(probf_hbm.at[idx16], val_v.at[pl.ds(h * 16, 16)], semA).wait()

    pltpu.async_copy(probf_hbm.at[pl.ds(word0, CH)], bufA, semA)
    pltpu.async_copy(probf_hbm.at[pl.ds(word0 + CH, CH)], bufB, semB)

    def row_body(r, recall_vec):
        r16 = jnp.full((16,), r, jnp.int32)
        v_b = plsc.load_gather(val_v, [r16])
        lab_b = plsc.load_gather(lab_v, [r16])
        lab = jnp.max(lab_b)
        s_lab = lab // 16
        rbase = word0 + r * NCLS
        last = r >= ROWS_W - 1

        pltpu.make_async_copy(probf_hbm.at[pl.ds(word0, CH)], bufA, semA).wait()
        cntA = _count_chunk(bufA, 0, lab, s_lab, v_b, iota)
        nextA = jnp.where(last, word0, rbase + NCLS)
        pltpu.async_copy(probf_hbm.at[pl.ds(nextA, CH)], bufA, semA)

        pltpu.make_async_copy(probf_hbm.at[pl.ds(word0, CH)], bufB, semB).wait()
        cntB = _count_chunk(bufB, SCH, lab, s_lab, v_b, iota)
        nextB = jnp.where(last, word0, rbase + NCLS + CH)
        pltpu.async_copy(probf_hbm.at[pl.ds(nextB, CH)], bufB, semB)

        rank = jnp.sum(cntA + cntB)
        hit = jnp.logical_and(rank < float(TOPK), lab != 0)
        return recall_vec + jnp.where(hit, 1.0, 0.0)

    recall_vec = lax.fori_loop(0, ROWS_W, row_body, jnp.zeros((16,), jnp.float32))
    pltpu.make_async_copy(probf_hbm.at[pl.ds(word0, CH)], bufA, semA).wait()
    pltpu.make_async_copy(probf_hbm.at[pl.ds(word0, CH)], bufB, semB).wait()

    out_v[...] = recall_vec
    pltpu.sync_copy(out_v, out_hbm.at[wid])


def kernel(prob, label):
    probf = prob.reshape(-1)
    mesh = plsc.VectorSubcoreMesh(core_axis_name="c", subcore_axis_name="s")
    run = functools.partial(
        pl.kernel,
        mesh=mesh,
        compiler_params=pltpu.CompilerParams(needs_layout_passes=False),
        out_type=jax.ShapeDtypeStruct((NWORK, 16), jnp.float32),
        scratch_types=[
            pltpu.VMEM((CH,), jnp.float32),
            pltpu.VMEM((CH,), jnp.float32),
            pltpu.VMEM((ROWS_W,), jnp.int32),
            pltpu.VMEM((ROWS_W,), jnp.float32),
            pltpu.VMEM((16,), jnp.float32),
            pltpu.SemaphoreType.DMA,
            pltpu.SemaphoreType.DMA,
        ],
    )(_reck_body)
    parts = run(probf, label)
    return parts[:, 0].sum() / jnp.float32(BATCH)


# speedup vs baseline: 4.6175x; 1.8525x over previous
import functools
import jax
import jax.numpy as jnp
from jax import lax
from jax.experimental import pallas as pl
from jax.experimental.pallas import tpu as pltpu
from jax.experimental.pallas import tpu_sc as plsc

TOPK = 5
BATCH = 1024
NCLS = 100000
NWORK = 32
ROWS_W = BATCH // NWORK
NGRP = ROWS_W // 8
CC = 4992
NCH = 20
SC_CH = CC // 16
TAIL0 = NCH * CC
TAILC = NCLS - TAIL0
TSL0 = TAIL0 // 16
NQ = NGRP * NCH
UNROLL = 24
NACC = 6


def _full(x):
    return jnp.full((16,), x, jnp.int32)


def _count_chunk(buf, s, b0s, lab, s_lab, v_b, iota):
    p_end = jnp.clip(s_lab - b0s, 0, SC_CH)
    g_start = jnp.clip(s_lab + 1 - b0s, 0, SC_CH)

    def mk_block(ge):
        def blk(i, accs):
            base = i * UNROLL
            accs = list(accs)
            for k in range(UNROLL):
                x = buf[s, pl.ds((base + k) * 16, 16)]
                cond = (x >= v_b) if ge else (x > v_b)
                accs[k % NACC] = accs[k % NACC] + jnp.where(cond, 1.0, 0.0)
            return tuple(accs)
        return blk

    def mk_one(ge):
        def one(i, acc):
            x = buf[s, pl.ds(i * 16, 16)]
            cond = (x >= v_b) if ge else (x > v_b)
            return acc + jnp.where(cond, 1.0, 0.0)
        return one

    zeros = jnp.zeros((16,), jnp.float32)
    accs = (zeros,) * NACC
    nfull = p_end // UNROLL
    accs = lax.fori_loop(0, nfull, mk_block(True), accs)
    rem = lax.fori_loop(nfull * UNROLL, p_end, mk_one(True), zeros)
    m = s_lab - b0s
    mc = jnp.clip(m, 0, SC_CH - 1)
    xm = buf[s, pl.ds(mc * 16, 16)]
    gidx = (b0s + mc) * 16 + iota
    mix = jnp.where(jnp.where(gidx < lab, xm >= v_b, xm > v_b), 1.0, 0.0)
    inr = jnp.logical_and(m >= 0, m < SC_CH)
    rem = rem + jnp.where(inr, mix, 0.0)
    g_align = jnp.minimum((g_start + UNROLL - 1) // UNROLL * UNROLL, SC_CH)
    rem = lax.fori_loop(g_start, g_align, mk_one(False), rem)
    accs = lax.fori_loop(g_align // UNROLL, SC_CH // UNROLL, mk_block(False), accs)
    total = rem
    for a in accs:
        total = total + a
    return total


def _reck_body(prob_hbm, label_hbm, out_hbm,
               bufA, bufB, vsl3, tail4, lab_v, vb_v, acc_v, out_v,
               semA, semB, semV):
    cid = lax.axis_index("c")
    sid = lax.axis_index("s")
    wid = sid * 2 + cid
    row0 = pl.multiple_of(wid * ROWS_W, 8)
    iota = lax.iota(jnp.int32, 16)
    bufs = (bufA, bufB)
    sems = (semA, semB)

    pltpu.sync_copy(label_hbm.at[pl.ds(row0, ROWS_W)], lab_v)

    handles = []
    for r in range(ROWS_W):
        lab_r = jnp.max(plsc.load_gather(lab_v, [_full(r)]))
        tcol = pl.multiple_of(
            jnp.minimum((lab_r // 128) * 128, TAIL0 - 128), 128)
        src = prob_hbm.at[pl.ds(pl.multiple_of(row0 + (r // 8) * 8, 8), 8),
                          pl.ds(tcol, 128)]
        handles.append(pltpu.async_copy(src, vsl3.at[r], semV))
    for g in range(NGRP):
        src = prob_hbm.at[pl.ds(pl.multiple_of(row0 + g * 8, 8), 8),
                          pl.ds(TAIL0, TAILC)]
        handles.append(pltpu.async_copy(src, tail4.at[g], semV))

    pltpu.async_copy(prob_hbm.at[pl.ds(row0, 8), pl.ds(0, CC)], bufA, semA)
    pltpu.async_copy(prob_hbm.at[pl.ds(row0, 8), pl.ds(CC, CC)], bufB, semB)

    for h in handles:
        h.wait()

    def group_body(g, recall_vec):
        for s in range(8):
            r = g * 8 + s
            lab_s = jnp.max(plsc.load_gather(lab_v, [_full(r)]))
            tcol = jnp.minimum((lab_s // 128) * 128, TAIL0 - 128)
            lane_t = jnp.clip(lab_s - tcol, 0, 127)
            v_tile = plsc.load_gather(vsl3, [_full(r), _full(s), _full(lane_t)])
            lane_w = jnp.clip(lab_s - TAIL0, 0, TAILC - 1)
            v_tail = plsc.load_gather(tail4, [_full(g), _full(s), _full(lane_w)])
            v_s = jnp.where(lab_s >= TAIL0, v_tail, v_tile)
            vb_v[s, pl.ds(0, 16)] = v_s
            acc_v[s, pl.ds(0, 16)] = jnp.zeros((16,), jnp.float32)

        def row_pass(buf, c):
            def srow(s, carry):
                r = g * 8 + s
                lab_s = jnp.max(plsc.load_gather(lab_v, [_full(r)]))
                s_lab = lab_s // 16
                v_b = plsc.load_gather(vb_v, [_full(s), iota])
                cnt = _count_chunk(buf, s, c * SC_CH, lab_s, s_lab, v_b, iota)
                acc_v[s, pl.ds(0, 16)] = acc_v[s, pl.ds(0, 16)] + cnt
                return carry
            return lax.fori_loop(0, 8, srow, 0)

        def chunk_body(t, carry):
            for b in range(2):
                c = t * 2 + b
                q = g * NCH + c
                pltpu.make_async_copy(
                    prob_hbm.at[pl.ds(row0, 8), pl.ds(0, CC)],
                    bufs[b], sems[b]).wait()
                row_pass(bufs[b], c)
                qn = q + 2
                valid = qn < NQ
                g2 = jnp.where(valid, qn // NCH, 0)
                c2 = jnp.where(valid, qn % NCH, 0)
                roff = pl.multiple_of(row0 + g2 * 8, 8)
                coff = pl.multiple_of(c2 * CC, 128)
                pltpu.async_copy(
                    prob_hbm.at[pl.ds(roff, 8), pl.ds(coff, CC)],
                    bufs[b], sems[b])
            return carry

        lax.fori_loop(0, NCH // 2, chunk_body, 0)

        def tail_row(s, carry):
            r = g * 8 + s
            lab_s = jnp.max(plsc.load_gather(lab_v, [_full(r)]))
            v_b = plsc.load_gather(vb_v, [_full(s), iota])
            cnt = jnp.zeros((16,), jnp.float32)
            for k in range(TAILC // 16):
                x = tail4[g, s, pl.ds(k * 16, 16)]
                gidx = (TSL0 + k) * 16 + iota
                cond = jnp.where(gidx < lab_s, x >= v_b, x > v_b)
                cnt = cnt + jnp.where(cond, 1.0, 0.0)
            acc_v[s, pl.ds(0, 16)] = acc_v[s, pl.ds(0, 16)] + cnt
            return carry
        lax.fori_loop(0, 8, tail_row, 0)

        def fin_row(s, rv):
            r = g * 8 + s
            lab_s = jnp.max(plsc.load_gather(lab_v, [_full(r)]))
            rank = jnp.sum(acc_v[s, pl.ds(0, 16)])
            hit = jnp.logical_and(rank < float(TOPK), lab_s != 0)
            return rv + jnp.where(hit, 1.0, 0.0)
        return lax.fori_loop(0, 8, fin_row, recall_vec)

    recall_vec = lax.fori_loop(0, NGRP, group_body,
                               jnp.zeros((16,), jnp.float32))

    pltpu.make_async_copy(prob_hbm.at[pl.ds(row0, 8), pl.ds(0, CC)],
                          bufA, semA).wait()
    pltpu.make_async_copy(prob_hbm.at[pl.ds(row0, 8), pl.ds(0, CC)],
                          bufB, semB).wait()

    out_v[...] = recall_vec
    pltpu.sync_copy(out_v, out_hbm.at[wid])


def kernel(prob, label):
    mesh = plsc.VectorSubcoreMesh(core_axis_name="c", subcore_axis_name="s")
    run = functools.partial(
        pl.kernel,
        mesh=mesh,
        compiler_params=pltpu.CompilerParams(
            needs_layout_passes=False, use_tc_tiling_on_sc=True),
        out_type=jax.ShapeDtypeStruct((NWORK, 16), jnp.float32),
        scratch_types=[
            pltpu.VMEM((8, CC), jnp.float32),
            pltpu.VMEM((8, CC), jnp.float32),
            pltpu.VMEM((ROWS_W, 8, 128), jnp.float32),
            pltpu.VMEM((NGRP, 8, TAILC), jnp.float32),
            pltpu.VMEM((ROWS_W,), jnp.int32),
            pltpu.VMEM((8, 16), jnp.float32),
            pltpu.VMEM((8, 16), jnp.float32),
            pltpu.VMEM((16,), jnp.float32),
            pltpu.SemaphoreType.DMA,
            pltpu.SemaphoreType.DMA,
            pltpu.SemaphoreType.DMA,
        ],
    )(_reck_body)
    parts = run(prob, label)
    return parts[:, 0].sum() / jnp.float32(BATCH)


# speedup vs baseline: 8.8880x; 1.9248x over previous
import functools
import jax
import jax.numpy as jnp
from jax import lax
from jax.experimental import pallas as pl
from jax.experimental.pallas import tpu as pltpu
from jax.experimental.pallas import tpu_sc as plsc

TOPK = 5
BATCH = 1024
NCLS = 100000
NSG = 8
NCB = 4
SPW = BATCH // NSG
CPW = NCLS // NCB
JC = 200
NCHK = CPW // JC
UNJ = 4


def _full(x):
    return jnp.full((16,), x, jnp.int32)


def _reck_body(probT_hbm, label_hbm, out_hbm,
               bufA, bufB, vsl3, lab_mine, vmine_v, v128_v, lab128_v,
               acc_v, out_v, comb_v, shv_sh, shacc_sh,
               semA, semB, semV):
    cid = lax.axis_index("c")
    sid = lax.axis_index("s")
    sg = cid * 4 + sid // 4
    cb = sid % 4
    s0 = pl.multiple_of(sg * SPW, 128)
    c0 = pl.multiple_of(cb * CPW, 8)
    iota = lax.iota(jnp.int32, 16)
    bufs = (bufA, bufB)
    sems = (semA, semB)

    sbase = s0 + cb * 32
    pltpu.sync_copy(label_hbm.at[pl.ds(sbase, 32)], lab_mine)
    handles = []
    labs = []
    for r in range(32):
        lab_r = jnp.max(plsc.load_gather(lab_mine, [_full(r)]))
        labs.append(lab_r)
        t8 = pl.multiple_of((lab_r // 8) * 8, 8)
        handles.append(pltpu.async_copy(
            probT_hbm.at[pl.ds(t8, 8), pl.ds(s0, SPW)], vsl3.at[r], semV))

    pltpu.async_copy(probT_hbm.at[pl.ds(c0, JC), pl.ds(s0, SPW)], bufA, semA)
    pltpu.async_copy(probT_hbm.at[pl.ds(c0 + JC, JC), pl.ds(s0, SPW)],
                     bufB, semB)
    for h in handles:
        h.wait()

    vv = [jnp.zeros((16,), jnp.float32), jnp.zeros((16,), jnp.float32)]
    for r in range(32):
        lab_r = labs[r]
        vr = plsc.load_gather(
            vsl3, [_full(r), _full(lab_r - (lab_r // 8) * 8),
                   _full(cb * 32 + r)])
        h = r // 16
        vv[h] = jnp.where(iota == (r - h * 16), vr, vv[h])
    vmine_v[pl.ds(0, 16)] = vv[0]
    vmine_v[pl.ds(16, 16)] = vv[1]

    pltpu.sync_copy(vmine_v, shv_sh.at[sid])
    plsc.subcore_barrier()
    grp0 = (sid // 4) * 4
    pltpu.sync_copy(shv_sh.at[pl.ds(grp0, 4)], v128_v)
    pltpu.sync_copy(label_hbm.at[pl.ds(s0, SPW)], lab128_v)

    for k in range(8):
        acc_v[k, pl.ds(0, 16)] = jnp.zeros((16,), jnp.float32)

    def labv(k):
        return lab128_v[pl.ds(k * 16, 16)]

    def bv(k):
        return lax.bitcast_convert_type(
            v128_v[k // 2, pl.ds((k % 2) * 16, 16)], jnp.int32)

    def do_chunk(c, buf):
        jb = c0 + c * JC
        ts = []
        accs = []
        for k in range(8):
            pred = labv(k) >= jb + JC
            ts.append(bv(k) - jnp.where(pred, 1, 0))
            accs.append(acc_v[k, pl.ds(0, 16)])

        def jbody(i, a):
            a = list(a)
            for u in range(UNJ):
                j = i * UNJ + u
                for k in range(8):
                    x = lax.bitcast_convert_type(
                        buf[j, pl.ds(k * 16, 16)], jnp.int32)
                    a[k] = a[k] + jnp.where(x > ts[k], 1.0, 0.0)
            return tuple(a)

        accs = lax.fori_loop(0, JC // UNJ, jbody, tuple(accs))
        for k in range(8):
            acc_v[k, pl.ds(0, 16)] = accs[k]

        insides = [jnp.logical_and(labv(k) >= jb, labv(k) < jb + JC)
                   for k in range(8)]
        anyv = insides[0]
        for k in range(1, 8):
            anyv = jnp.logical_or(anyv, insides[k])
        any_in = jnp.max(jnp.where(anyv, 1, 0)) > 0

        @pl.when(any_in)
        def _corr():
            def cbody(j, a):
                a = list(a)
                jg = _full(jb + j)
                for k in range(8):
                    x = lax.bitcast_convert_type(
                        buf[j, pl.ds(k * 16, 16)], jnp.int32)
                    hitc = jnp.logical_and(
                        jnp.logical_and(x == bv(k), jg < labv(k)), insides[k])
                    a[k] = a[k] + jnp.where(hitc, 1.0, 0.0)
                return tuple(a)
            cz = tuple(jnp.zeros((16,), jnp.float32) for _ in range(8))
            cs = lax.fori_loop(0, JC, cbody, cz)
            for k in range(8):
                acc_v[k, pl.ds(0, 16)] = acc_v[k, pl.ds(0, 16)] + cs[k]

    def pair_body(t, carry):
        for b in range(2):
            c = t * 2 + b
            pltpu.make_async_copy(
                probT_hbm.at[pl.ds(c0, JC), pl.ds(s0, SPW)],
                bufs[b], sems[b]).wait()
            do_chunk(c, bufs[b])
            cn = c + 2
            coff = pl.multiple_of(
                c0 + jnp.where(cn < NCHK, cn, 0) * JC, 8)
            pltpu.async_copy(
                probT_hbm.at[pl.ds(coff, JC), pl.ds(s0, SPW)],
                bufs[b], sems[b])
        return carry

    lax.fori_loop(0, (NCHK - 1) // 2, pair_body, 0)
    pltpu.make_async_copy(probT_hbm.at[pl.ds(c0, JC), pl.ds(s0, SPW)],
                          bufA, semA).wait()
    do_chunk(NCHK - 1, bufA)
    pltpu.make_async_copy(probT_hbm.at[pl.ds(c0, JC), pl.ds(s0, SPW)],
                          bufB, semB).wait()

    for k in range(8):
        out_v[k, pl.ds(0, 16)] = acc_v[k, pl.ds(0, 16)]
    pltpu.sync_copy(out_v, shacc_sh.at[sid])
    plsc.subcore_barrier()

    @pl.when(cb == 0)
    def _finish():
        pltpu.sync_copy(shacc_sh.at[pl.ds(grp0, 4)], comb_v)
        hits = jnp.zeros((16,), jnp.float32)
        for k in range(8):
            rank = (comb_v[0, k, pl.ds(0, 16)]
                    + comb_v[1, k, pl.ds(0, 16)]
                    + comb_v[2, k, pl.ds(0, 16)]
                    + comb_v[3, k, pl.ds(0, 16)])
            ok = jnp.logical_and(rank < float(TOPK), labv(k) != 0)
            hits = hits + jnp.where(ok, 1.0, 0.0)
        out_v[0, pl.ds(0, 16)] = hits
        pltpu.sync_copy(out_v.at[0], out_hbm.at[sg])


def kernel(prob, label):
    probT = prob.T
    mesh = plsc.VectorSubcoreMesh(core_axis_name="c", subcore_axis_name="s")
    run = functools.partial(
        pl.kernel,
        mesh=mesh,
        compiler_params=pltpu.CompilerParams(
            needs_layout_passes=False, use_tc_tiling_on_sc=True),
        out_type=jax.ShapeDtypeStruct((NSG, 16), jnp.float32),
        scratch_types=[
            pltpu.VMEM((JC, SPW), jnp.float32),
            pltpu.VMEM((JC, SPW), jnp.float32),
            pltpu.VMEM((32, 8, SPW), jnp.float32),
            pltpu.VMEM((32,), jnp.int32),
            pltpu.VMEM((32,), jnp.float32),
            pltpu.VMEM((4, 32), jnp.float32),
            pltpu.VMEM((SPW,), jnp.int32),
            pltpu.VMEM((8, 16), jnp.float32),
            pltpu.VMEM((8, 16), jnp.float32),
            pltpu.VMEM((4, 8, 16), jnp.float32),
            pltpu.VMEM_SHARED((16, 32), jnp.float32),
            pltpu.VMEM_SHARED((16, 8, 16), jnp.float32),
            pltpu.SemaphoreType.DMA,
            pltpu.SemaphoreType.DMA,
            pltpu.SemaphoreType.DMA,
        ],
    )(_reck_body)
    parts = run(probT, label)
    return parts.sum() / jnp.float32(BATCH)
